# baseline (speedup 1.0000x reference)
import dataclasses
import functools

import jax
import jax.numpy as jnp
from jax import lax
from jax.experimental import pallas as pl
from jax.experimental.pallas import tpu as pltpu
from jax.experimental.pallas import tpu_sc as plsc

_N_CAT = 64
_NUM_NODE = 2048
_NUM_REL = 4
_TOTAL = _NUM_NODE * _NUM_NODE * _NUM_REL
_N_OUT = _NUM_NODE * _NUM_NODE
_BLK = 32768


def _pair_table(edge_cat_vectors, edge_cat_vectors_t, query_row, query_col):
    def body(ecv_ref, ecvt_ref, qr_ref, qc_ref, p_ref):
        r32 = lambda x: x.astype(jnp.bfloat16).astype(jnp.float32)
        logit_c = jnp.sum(
            r32(ecv_ref[...]) * r32(qr_ref[...]), axis=1, keepdims=True
        )
        logit_r = jnp.sum(
            r32(ecvt_ref[...]) * r32(qc_ref[...]), axis=0, keepdims=True
        )
        m = jnp.max(logit_c)

        e_c = jnp.exp(logit_c - m)
        row = lax.broadcasted_iota(jnp.int32, e_c.shape, 0)
        e_c = jnp.where(row == 0, 0.0, e_c)
        ca_c = e_c / jnp.sum(e_c)

        e_r = jnp.exp(logit_r - m)
        col = lax.broadcasted_iota(jnp.int32, e_r.shape, 1)
        e_r = jnp.where(col == 0, 0.0, e_r)
        ca_r = e_r / jnp.sum(e_r)

        p_ref[...] = jnp.broadcast_to(ca_c, (_N_CAT, _N_CAT)) + jnp.broadcast_to(
            ca_r, (_N_CAT, _N_CAT)
        )

    return pl.pallas_call(
        body,
        out_shape=jax.ShapeDtypeStruct((_N_CAT, _N_CAT), jnp.float32),
    )(edge_cat_vectors, edge_cat_vectors_t, query_row, query_col)


def _sc_lookup(ptab_flat, idx_flat):
    mesh = plsc.VectorSubcoreMesh(core_axis_name="c", subcore_axis_name="s")
    cp = pltpu.CompilerParams()
    if "needs_layout_passes" in pltpu.CompilerParams.__dataclass_fields__:
        cp = dataclasses.replace(cp, needs_layout_passes=False)

    @functools.partial(
        pl.kernel,
        mesh=mesh,
        out_type=jax.ShapeDtypeStruct((_N_OUT,), jnp.float32),
        scratch_types=[pltpu.VMEM((_N_CAT * _N_CAT,), jnp.float32)],
        compiler_params=cp,
    )
    def k(ptab_hbm, idx_hbm, out_hbm, ptab_v):
        pltpu.sync_copy(ptab_hbm, ptab_v)
        iota4 = lax.iota(jnp.int32, 16) * 4

        def body(in_v, out_v):
            @pl.loop(0, _BLK // 64)
            def _(j):
                i0 = j * 64 + iota4
                m0 = plsc.load_gather(in_v, [i0])
                m1 = plsc.load_gather(in_v, [i0 + 1])
                m2 = plsc.load_gather(in_v, [i0 + 2])
                m3 = plsc.load_gather(in_v, [i0 + 3])
                p1 = (m0 << 6) | m1
                p2 = (m2 << 6) | m3
                t = plsc.load_gather(ptab_v, [p1]) + plsc.load_gather(
                    ptab_v, [p2]
                )
                out_v[pl.ds(j * 16, 16)] = t

        pltpu.emit_pipeline(
            body,
            grid=(_TOTAL // _BLK,),
            in_specs=[pl.BlockSpec((_BLK,), lambda i: (i,))],
            out_specs=[pl.BlockSpec((_BLK // 4,), lambda i: (i,))],
            core_axis_name=("c", "s"),
            dimension_semantics=(pltpu.PARALLEL,),
        )(idx_hbm, out_hbm)

    return k(ptab_flat, idx_flat)


def kernel(edge_cat_vectors, query, cat_matrix):
    ptab = _pair_table(
        edge_cat_vectors,
        edge_cat_vectors.T,
        query.reshape(1, -1),
        query.reshape(-1, 1),
    )
    idx_flat = cat_matrix.astype(jnp.int32).reshape(-1)
    out = _sc_lookup(ptab.reshape(-1), idx_flat)
    return out.reshape(_NUM_NODE, _NUM_NODE)


# speedup vs baseline: 23.7937x; 23.7937x over previous
import dataclasses
import functools

import jax
import jax.numpy as jnp
from jax import lax
from jax.experimental import pallas as pl
from jax.experimental.pallas import tpu as pltpu
from jax.experimental.pallas import tpu_sc as plsc

_N_CAT = 64
_NUM_NODE = 2048
_NUM_REL = 4
_TOTAL = _NUM_NODE * _NUM_NODE * _NUM_REL
_N_OUT = _NUM_NODE * _NUM_NODE
_BLK = 16384


def _pair_table(edge_cat_vectors, edge_cat_vectors_t, query_row, query_col):
    def body(ecv_ref, ecvt_ref, qr_ref, qc_ref, p_ref):
        r32 = lambda x: x.astype(jnp.bfloat16).astype(jnp.float32)
        logit_c = jnp.sum(
            r32(ecv_ref[...]) * r32(qr_ref[...]), axis=1, keepdims=True
        )
        logit_r = jnp.sum(
            r32(ecvt_ref[...]) * r32(qc_ref[...]), axis=0, keepdims=True
        )
        m = jnp.max(logit_c)

        e_c = jnp.exp(logit_c - m)
        row = lax.broadcasted_iota(jnp.int32, e_c.shape, 0)
        e_c = jnp.where(row == 0, 0.0, e_c)
        ca_c = e_c / jnp.sum(e_c)

        e_r = jnp.exp(logit_r - m)
        col = lax.broadcasted_iota(jnp.int32, e_r.shape, 1)
        e_r = jnp.where(col == 0, 0.0, e_r)
        ca_r = e_r / jnp.sum(e_r)

        p_ref[...] = jnp.broadcast_to(ca_c, (_N_CAT, _N_CAT)) + jnp.broadcast_to(
            ca_r, (_N_CAT, _N_CAT)
        )

    return pl.pallas_call(
        body,
        out_shape=jax.ShapeDtypeStruct((_N_CAT, _N_CAT), jnp.float32),
    )(edge_cat_vectors, edge_cat_vectors_t, query_row, query_col)


def _sc_lookup(ptab_flat, quad_flat):
    mesh = plsc.VectorSubcoreMesh(core_axis_name="c", subcore_axis_name="s")
    cp = pltpu.CompilerParams()
    if "needs_layout_passes" in pltpu.CompilerParams.__dataclass_fields__:
        cp = dataclasses.replace(cp, needs_layout_passes=False)

    @functools.partial(
        pl.kernel,
        mesh=mesh,
        out_type=jax.ShapeDtypeStruct((_N_OUT,), jnp.float32),
        scratch_types=[pltpu.VMEM((_N_CAT * _N_CAT,), jnp.float32)],
        compiler_params=cp,
    )
    def k(ptab_hbm, quad_hbm, out_hbm, ptab_v):
        pltpu.sync_copy(ptab_hbm, ptab_v)

        def body(in_v, out_v):
            @pl.loop(0, _BLK // 64)
            def _(j):
                for r in range(4):
                    o = j * 64 + r * 16
                    v = in_v[pl.ds(o, 16)]
                    p1 = v & 0xFFF
                    p2 = lax.shift_right_logical(v, 12)
                    t = plsc.load_gather(ptab_v, [p1]) + plsc.load_gather(
                        ptab_v, [p2]
                    )
                    out_v[pl.ds(o, 16)] = t

        pltpu.emit_pipeline(
            body,
            grid=(_N_OUT // _BLK,),
            in_specs=[pl.BlockSpec((_BLK,), lambda i: (i,))],
            out_specs=[pl.BlockSpec((_BLK,), lambda i: (i,))],
            core_axis_name=("c", "s"),
            dimension_semantics=(pltpu.PARALLEL,),
        )(quad_hbm, out_hbm)

    return k(ptab_flat, quad_flat)


def kernel(edge_cat_vectors, query, cat_matrix):
    ptab = _pair_table(
        edge_cat_vectors,
        edge_cat_vectors.T,
        query.reshape(1, -1),
        query.reshape(-1, 1),
    )
    cm = cat_matrix.astype(jnp.int32)
    quad = (
        cm[:, :, 0]
        | (cm[:, :, 1] << 6)
        | (cm[:, :, 2] << 12)
        | (cm[:, :, 3] << 18)
    ).reshape(-1)
    out = _sc_lookup(ptab.reshape(-1), quad)
    return out.reshape(_NUM_NODE, _NUM_NODE)


# speedup vs baseline: 35.9558x; 1.5111x over previous
import dataclasses
import functools

import jax
import jax.numpy as jnp
from jax import lax
from jax.experimental import pallas as pl
from jax.experimental.pallas import tpu as pltpu
from jax.experimental.pallas import tpu_sc as plsc

_N_CAT = 64
_NUM_NODE = 2048
_NUM_REL = 4
_TOTAL = _NUM_NODE * _NUM_NODE * _NUM_REL
_N_OUT = _NUM_NODE * _NUM_NODE
_BLK = 16384


def _pair_table(edge_cat_vectors, edge_cat_vectors_t, query_row, query_col):
    def body(ecv_ref, ecvt_ref, qr_ref, qc_ref, p_ref):
        r32 = lambda x: x.astype(jnp.bfloat16).astype(jnp.float32)
        logit_c = jnp.sum(
            r32(ecv_ref[...]) * r32(qr_ref[...]), axis=1, keepdims=True
        )
        logit_r = jnp.sum(
            r32(ecvt_ref[...]) * r32(qc_ref[...]), axis=0, keepdims=True
        )
        m = jnp.max(logit_c)

        e_c = jnp.exp(logit_c - m)
        row = lax.broadcasted_iota(jnp.int32, e_c.shape, 0)
        e_c = jnp.where(row == 0, 0.0, e_c)
        ca_c = e_c / jnp.sum(e_c)

        e_r = jnp.exp(logit_r - m)
        col = lax.broadcasted_iota(jnp.int32, e_r.shape, 1)
        e_r = jnp.where(col == 0, 0.0, e_r)
        ca_r = e_r / jnp.sum(e_r)

        p_ref[...] = jnp.broadcast_to(ca_c, (_N_CAT, _N_CAT)) + jnp.broadcast_to(
            ca_r, (_N_CAT, _N_CAT)
        )

    return pl.pallas_call(
        body,
        out_shape=jax.ShapeDtypeStruct((_N_CAT, _N_CAT), jnp.float32),
    )(edge_cat_vectors, edge_cat_vectors_t, query_row, query_col)


def _sc_lookup(ptab_flat, quad_flat):
    mesh = plsc.VectorSubcoreMesh(core_axis_name="c", subcore_axis_name="s")
    cp = pltpu.CompilerParams()
    if "needs_layout_passes" in pltpu.CompilerParams.__dataclass_fields__:
        cp = dataclasses.replace(cp, needs_layout_passes=False)

    @functools.partial(
        pl.kernel,
        mesh=mesh,
        out_type=jax.ShapeDtypeStruct((_N_OUT,), jnp.float32),
        scratch_types=[pltpu.VMEM((_N_CAT * _N_CAT,), jnp.float32)],
        compiler_params=cp,
    )
    def k(ptab_hbm, quad_hbm, out_hbm, ptab_v):
        pltpu.sync_copy(ptab_hbm, ptab_v)

        def body(in_v, out_v):
            @plsc.parallel_loop(0, _BLK, 64, unroll=4)
            def _(j):
                for r in range(4):
                    o = j + r * 16
                    v = in_v[pl.ds(o, 16)]
                    p1 = v & 0xFFF
                    p2 = lax.shift_right_logical(v, 12)
                    t = plsc.load_gather(ptab_v, [p1]) + plsc.load_gather(
                        ptab_v, [p2]
                    )
                    out_v[pl.ds(o, 16)] = t

        pltpu.emit_pipeline(
            body,
            grid=(_N_OUT // _BLK,),
            in_specs=[pl.BlockSpec((_BLK,), lambda i: (i,))],
            out_specs=[pl.BlockSpec((_BLK,), lambda i: (i,))],
            core_axis_name=("c", "s"),
            dimension_semantics=(pltpu.PARALLEL,),
        )(quad_hbm, out_hbm)

    return k(ptab_flat, quad_flat)


def kernel(edge_cat_vectors, query, cat_matrix):
    ptab = _pair_table(
        edge_cat_vectors,
        edge_cat_vectors.T,
        query.reshape(1, -1),
        query.reshape(-1, 1),
    )
    cm = cat_matrix.astype(jnp.int32)
    quad = (
        cm[:, :, 0]
        | (cm[:, :, 1] << 6)
        | (cm[:, :, 2] << 12)
        | (cm[:, :, 3] << 18)
    ).reshape(-1)
    out = _sc_lookup(ptab.reshape(-1), quad)
    return out.reshape(_NUM_NODE, _NUM_NODE)
